# baseline (speedup 1.0000x reference)
import functools

import numpy as np

import jax
import jax.numpy as jnp
from jax import lax
from jax.experimental import pallas as pl
from jax.experimental.pallas import tpu as pltpu
from jax.experimental.pallas import tpu_sc as plsc

L = 16
NC = 2
NS = 16
R = 128


def _sc_pooling(Bn, T, D):
    n = T // Bn
    half = n // 2
    NB = half // R
    DV = D // L
    nseg_per_core = Bn // NC

    mesh = plsc.VectorSubcoreMesh(
        core_axis_name="c", subcore_axis_name="s", num_cores=NC,
        num_subcores=NS)

    @functools.partial(
        pl.kernel,
        out_type=jax.ShapeDtypeStruct((Bn, D), jnp.float32),
        mesh=mesh,
        compiler_params=pltpu.CompilerParams(needs_layout_passes=False),
        scratch_types=[
            pltpu.VMEM((D + 2 * L,), jnp.float32),
            pltpu.VMEM((2, R, D // 2), jnp.float32),
            pltpu.VMEM((R,), jnp.float32),
            pltpu.VMEM((D + L,), jnp.float32),
            pltpu.VMEM((D + L,), jnp.float32),
            pltpu.VMEM((D,), jnp.float32),
            pltpu.VMEM_SHARED((NS * 512,), jnp.float32),
            pltpu.SemaphoreType.DMA,
            pltpu.SemaphoreType.DMA,
        ],
    )
    def call(x_hbm, params_hbm, out_hbm, params_v, xbuf, logits_v,
             acc_ref, pacc_v, outbuf, shex, sem0, sem1):
        c = lax.axis_index("c")
        s = lax.axis_index("s")
        seg = c * nseg_per_core + s // 2
        h = s % 2
        base = seg * n + h * half

        pltpu.sync_copy(params_hbm, params_v)
        lane = lax.iota(jnp.int32, L)
        pcoef_v = params_v[pl.ds(D, L)]
        pcoef = jnp.sum(jnp.where(lane == seg, pcoef_v, 0.0))
        bias = params_v[pl.ds(D + Bn, L)][0]
        lanef = lane.astype(jnp.float32)
        zero16 = jnp.zeros((L,), jnp.float32)
        for j in range(DV):
            acc_ref[pl.ds(L * j, L)] = zero16

        sems = (sem0, sem1)
        pltpu.async_copy(x_hbm.at[pl.ds(base, R), :], xbuf.at[0], sem0)
        pltpu.async_copy(x_hbm.at[pl.ds(base + R, R), :], xbuf.at[1], sem1)

        def process(blk, b, m, s_v):
            xb = xbuf.at[b]
            toff = (h * half + blk * R).astype(jnp.float32)

            def grp_a(g, bm_v):
                rb = g * L
                dv = zero16
                for q in range(0, L, 8):
                    p = [zero16] * 8
                    for j2 in range(DV // 2):
                        wa = params_v[pl.ds(2 * L * j2, L)]
                        wb = params_v[pl.ds(2 * L * j2 + L, L)]
                        for r in range(8):
                            va, vb = plsc.unpack(
                                plsc.bitcast(
                                    xb[rb + q + r, pl.ds(L * j2, L)],
                                    jnp.bfloat16),
                                format=plsc.PackFormat.INTERLEAVED)
                            p[r] = ((p[r] + va * wa) + vb * wb)
                    for r in range(8):
                        dv = jnp.where(lane == q + r, jnp.sum(p[r]), dv)
                tloc = lanef + (toff + (g * L).astype(jnp.float32))
                lv = dv + tloc * pcoef + bias
                logits_v[pl.ds(g * L, L)] = lv
                return jnp.maximum(bm_v, lv)

            bmax_v = lax.fori_loop(0, R // L, grp_a,
                                   jnp.full((L,), -1e30, jnp.float32))
            m_new = jnp.maximum(m, jnp.max(bmax_v))
            resc = jnp.exp(jnp.full((L,), m - m_new))
            s_v = s_v * resc
            for j in range(DV):
                acc_ref[pl.ds(L * j, L)] = acc_ref[pl.ds(L * j, L)] * resc

            def grp_b(g, sv):
                rb = g * L
                ev = jnp.exp(logits_v[pl.ds(g * L, L)] - m_new)
                for q in range(0, L, 8):
                    esp = [jnp.full((L,), ev[q + r]) for r in range(8)]
                    for jc in range(0, DV // 2, 4):
                        contrib = []
                        for j2 in range(jc, jc + 4):
                            pra = []
                            prb = []
                            for r in range(8):
                                va, vb = plsc.unpack(
                                    plsc.bitcast(
                                        xb[rb + q + r, pl.ds(L * j2, L)],
                                        jnp.bfloat16),
                                    format=plsc.PackFormat.INTERLEAVED)
                                pra.append(esp[r] * va)
                                prb.append(esp[r] * vb)
                            ca = ((pra[0] + pra[1]) + (pra[2] + pra[3])) + \
                                 ((pra[4] + pra[5]) + (pra[6] + pra[7]))
                            cb = ((prb[0] + prb[1]) + (prb[2] + prb[3])) + \
                                 ((prb[4] + prb[5]) + (prb[6] + prb[7]))
                            contrib.append((ca, cb))
                        for j2 in range(jc, jc + 4):
                            ca, cb = contrib[j2 - jc]
                            plsc.addupdate(
                                acc_ref.at[pl.ds(2 * L * j2, L)], ca)
                            plsc.addupdate(
                                acc_ref.at[pl.ds(2 * L * j2 + L, L)], cb)
                return sv + ev

            s_v = lax.fori_loop(0, R // L, grp_b, s_v)
            return m_new, s_v

        def pair(p, carry):
            m, s_v = carry
            for sub in range(2):
                blk = 2 * p + sub
                wait_src = x_hbm.at[pl.ds(0, R), :]
                pltpu.make_async_copy(wait_src, xbuf.at[sub],
                                      sems[sub]).wait()
                m, s_v = process(blk, sub, m, s_v)

                @pl.when(blk + 2 < NB)
                def _():
                    pltpu.async_copy(
                        x_hbm.at[pl.ds(base + (blk + 2) * R, R), :],
                        xbuf.at[sub], sems[sub])
            return m, s_v

        m, s_v = lax.fori_loop(
            0, NB // 2, pair,
            (jnp.float32(-1e30), zero16))

        s_loc = jnp.sum(s_v)
        st = jnp.where(lane == 0, m, jnp.where(lane == 1, s_loc, 0.0))
        acc_ref[pl.ds(D, L)] = st
        pltpu.sync_copy(acc_ref, shex.at[pl.ds(s * 512, D + L)])
        plsc.subcore_barrier()

        @pl.when(h == 0)
        def _():
            pltpu.sync_copy(shex.at[pl.ds((s + 1) * 512, D + L)], pacc_v)
            pst = pacc_v[pl.ds(D, L)]
            m2 = pst[0]
            s2 = pst[1]
            mf = jnp.maximum(m, m2)
            a1 = jnp.exp(jnp.full((L,), m - mf))
            a2 = jnp.exp(jnp.full((L,), m2 - mf))
            inv = 1.0 / (a1 * s_loc + a2 * s2)
            for j in range(DV):
                outbuf[pl.ds(L * j, L)] = (
                    acc_ref[pl.ds(L * j, L)] * a1
                    + pacc_v[pl.ds(L * j, L)] * a2) * inv
            pltpu.sync_copy(outbuf, out_hbm.at[seg])

    return call


def kernel(x, W, b, slices):
    T, D = x.shape
    Bn = slices.shape[0]
    q_perm = np.arange(D).reshape(D // 32, 16, 2).transpose(0, 2, 1)
    q_perm = q_perm.reshape(-1)
    inv_q = np.argsort(q_perm)
    wx = W[0, 1:][q_perm]
    pcoef = W[0, 0] / slices.astype(jnp.float32)
    pad = jnp.zeros((2 * L - Bn - 1,), jnp.float32)
    params = jnp.concatenate([wx, pcoef, b.astype(jnp.float32), pad])
    xpacked = lax.bitcast_convert_type(
        x.astype(jnp.bfloat16).reshape(T, D // 2, 2), jnp.float32)
    out_slots = _sc_pooling(Bn, T, D)(xpacked, params)
    return jnp.take(out_slots, inv_q, axis=1)


# speedup vs baseline: 4.6721x; 4.6721x over previous
import functools

import jax
import jax.numpy as jnp
from jax import lax
from jax.experimental import pallas as pl
from jax.experimental.pallas import tpu as pltpu
from jax.experimental.pallas import tpu_sc as plsc

L = 16
NC = 2
NS = 16
R = 128


def _sc_pooling(Bn, T, D, F):
    n = T // Bn
    half = F // 2
    NB = half // R
    DV = D // L
    nseg_per_core = Bn // NC

    mesh = plsc.VectorSubcoreMesh(
        core_axis_name="c", subcore_axis_name="s", num_cores=NC,
        num_subcores=NS)

    @functools.partial(
        pl.kernel,
        out_type=jax.ShapeDtypeStruct((Bn, D + L), jnp.float32),
        mesh=mesh,
        compiler_params=pltpu.CompilerParams(needs_layout_passes=False),
        scratch_types=[
            pltpu.VMEM((D + 2 * L,), jnp.float32),
            pltpu.VMEM((2, R, D), jnp.float32),
            pltpu.VMEM((R,), jnp.float32),
            pltpu.VMEM((D + L,), jnp.float32),
            pltpu.VMEM((D + L,), jnp.float32),
            pltpu.VMEM((D + L,), jnp.float32),
            pltpu.VMEM_SHARED((NS * 512,), jnp.float32),
            pltpu.SemaphoreType.DMA,
            pltpu.SemaphoreType.DMA,
        ],
    )
    def call(x_hbm, params_hbm, out_hbm, params_v, xbuf, logits_v,
             acc_ref, pacc_v, outbuf, shex, sem0, sem1):
        c = lax.axis_index("c")
        s = lax.axis_index("s")
        seg = c * nseg_per_core + s // 2
        h = s % 2
        base = seg * n + h * half

        pltpu.sync_copy(params_hbm, params_v)
        lane = lax.iota(jnp.int32, L)
        pcoef_v = params_v[pl.ds(D, L)]
        pcoef = jnp.sum(jnp.where(lane == seg, pcoef_v, 0.0))
        bias = params_v[pl.ds(D + Bn, L)][0]
        lanef = lane.astype(jnp.float32)
        zero16 = jnp.zeros((L,), jnp.float32)
        for j in range(DV):
            acc_ref[pl.ds(L * j, L)] = zero16

        sems = (sem0, sem1)
        pltpu.async_copy(x_hbm.at[pl.ds(base, R), :], xbuf.at[0], sem0)
        pltpu.async_copy(x_hbm.at[pl.ds(base + R, R), :], xbuf.at[1], sem1)

        def process(blk, b, m, s_v):
            xb = xbuf.at[b]
            toff = (h * half + blk * R).astype(jnp.float32)

            def grp_a(g, bm_v):
                rb = g * L
                dv = zero16
                for q in range(0, L, 8):
                    p = [zero16] * 8
                    for j in range(DV):
                        w = params_v[pl.ds(L * j, L)]
                        for r in range(8):
                            p[r] = p[r] + xb[rb + q + r,
                                             pl.ds(L * j, L)] * w
                    for r in range(8):
                        dv = jnp.where(lane == q + r, jnp.sum(p[r]), dv)
                tloc = lanef + (toff + (g * L).astype(jnp.float32))
                lv = dv + tloc * pcoef + bias
                logits_v[pl.ds(g * L, L)] = lv
                return jnp.maximum(bm_v, lv)

            bmax_v = lax.fori_loop(0, R // L, grp_a,
                                   jnp.full((L,), -1e30, jnp.float32))
            m_new = jnp.maximum(m, jnp.max(bmax_v))
            resc = jnp.exp(jnp.full((L,), m - m_new))
            s_v = s_v * resc
            for j in range(DV):
                acc_ref[pl.ds(L * j, L)] = acc_ref[pl.ds(L * j, L)] * resc

            def grp_b(g, sv):
                rb = g * L
                ev = jnp.exp(logits_v[pl.ds(g * L, L)] - m_new)
                for q in range(0, L, 8):
                    esp = [jnp.full((L,), ev[q + r]) for r in range(8)]
                    for jc in range(0, DV, 8):
                        contrib = []
                        for j in range(jc, jc + 8):
                            pr = [esp[r] * xb[rb + q + r, pl.ds(L * j, L)]
                                  for r in range(8)]
                            s01 = pr[0] + pr[1]
                            s23 = pr[2] + pr[3]
                            s45 = pr[4] + pr[5]
                            s67 = pr[6] + pr[7]
                            contrib.append((s01 + s23) + (s45 + s67))
                        for j in range(jc, jc + 8):
                            plsc.addupdate(acc_ref.at[pl.ds(L * j, L)],
                                           contrib[j - jc])
                return sv + ev

            s_v = lax.fori_loop(0, R // L, grp_b, s_v)
            return m_new, s_v

        def pair(p, carry):
            m, s_v = carry
            for sub in range(2):
                blk = 2 * p + sub
                wait_src = x_hbm.at[pl.ds(0, R), :]
                pltpu.make_async_copy(wait_src, xbuf.at[sub],
                                      sems[sub]).wait()
                m, s_v = process(blk, sub, m, s_v)

                @pl.when(blk + 2 < NB)
                def _():
                    pltpu.async_copy(
                        x_hbm.at[pl.ds(base + (blk + 2) * R, R), :],
                        xbuf.at[sub], sems[sub])
            return m, s_v

        m, s_v = lax.fori_loop(
            0, NB // 2, pair,
            (jnp.float32(-1e30), zero16))

        s_loc = jnp.sum(s_v)
        st = jnp.where(lane == 0, m, jnp.where(lane == 1, s_loc, 0.0))
        acc_ref[pl.ds(D, L)] = st
        pltpu.sync_copy(acc_ref, shex.at[pl.ds(s * 512, D + L)])
        plsc.subcore_barrier()

        @pl.when(h == 0)
        def _():
            pltpu.sync_copy(shex.at[pl.ds((s + 1) * 512, D + L)], pacc_v)
            pst = pacc_v[pl.ds(D, L)]
            m2 = pst[0]
            s2 = pst[1]
            mf = jnp.maximum(m, m2)
            a1 = jnp.exp(jnp.full((L,), m - mf))
            a2 = jnp.exp(jnp.full((L,), m2 - mf))
            for j in range(DV):
                outbuf[pl.ds(L * j, L)] = (
                    acc_ref[pl.ds(L * j, L)] * a1
                    + pacc_v[pl.ds(L * j, L)] * a2)
            sp = a1 * s_loc + a2 * s2
            stv = jnp.where(lane == 0, mf, jnp.where(lane == 1, sp[0], 0.0))
            outbuf[pl.ds(D, L)] = stv
            pltpu.sync_copy(outbuf, out_hbm.at[seg])

    return call


def _tc_tail(Bn, T, D, F):
    n = T // Bn
    RT = n - F

    def body(x_ref, w_ref, p2_ref, stats_ref, acc_ref):
        i = pl.program_id(0)
        xb = x_ref[...]
        wv = w_ref[...]
        pr = p2_ref[pl.ds(i, 1), :]
        pc = pr[0:1, 0:1]
        bb = pr[0:1, 1:2]
        tloc = jax.lax.broadcasted_iota(
            jnp.int32, (RT, 1), 0).astype(jnp.float32) + float(F)
        lg = jnp.dot(xb, wv,
                     preferred_element_type=jnp.float32) + tloc * pc + bb
        m = jnp.max(lg)
        e = jnp.exp(lg - m)
        col = jax.lax.broadcasted_iota(jnp.int32, (1, 128), 1)
        stats_ref[pl.ds(i, 1), :] = jnp.where(
            col == 0, m, jnp.where(col == 1, jnp.sum(e), 0.0))
        acc_ref[pl.ds(i, 1), :] = jnp.sum(xb * e, axis=0, keepdims=True)

    return pl.pallas_call(
        body,
        grid=(Bn,),
        in_specs=[
            pl.BlockSpec((RT, D), lambda i: (i * (n // RT) + F // RT, 0)),
            pl.BlockSpec((D, 1), lambda i: (0, 0)),
            pl.BlockSpec((Bn, 128), lambda i: (0, 0)),
        ],
        out_specs=[
            pl.BlockSpec((Bn, 128), lambda i: (0, 0)),
            pl.BlockSpec((Bn, D), lambda i: (0, 0)),
        ],
        out_shape=[
            jax.ShapeDtypeStruct((Bn, 128), jnp.float32),
            jax.ShapeDtypeStruct((Bn, D), jnp.float32),
        ],
    )


def kernel(x, W, b, slices):
    T, D = x.shape
    Bn = slices.shape[0]
    n = T // Bn
    F = n // 2
    wx = W[0, 1:]
    pcoef = W[0, 0] / slices.astype(jnp.float32)
    pad = jnp.zeros((2 * L - Bn - 1,), jnp.float32)
    params = jnp.concatenate([wx, pcoef, b.astype(jnp.float32), pad])

    params2 = jnp.pad(
        jnp.stack([pcoef, jnp.full((Bn,), b[0], jnp.float32)], axis=1),
        ((0, 0), (0, 126)))
    sc_out = _sc_pooling(Bn, T, D, F)(x, params)
    stats_tc, acc_tc = _tc_tail(Bn, T, D, F)(x, wx[:, None], params2)

    acc_sc = sc_out[:, :D]
    m_sc = sc_out[:, D]
    s_sc = sc_out[:, D + 1]
    m2 = stats_tc[:, 0]
    s2 = stats_tc[:, 1]
    mf = jnp.maximum(m_sc, m2)
    a1 = jnp.exp(m_sc - mf)
    a2 = jnp.exp(m2 - mf)
    denom = s_sc * a1 + s2 * a2
    return (acc_sc * a1[:, None] + acc_tc * a2[:, None]) / denom[:, None]


# speedup vs baseline: 4.7254x; 1.0114x over previous
import functools

import jax
import jax.numpy as jnp
from jax import lax
from jax.experimental import pallas as pl
from jax.experimental.pallas import tpu as pltpu
from jax.experimental.pallas import tpu_sc as plsc

L = 16
NC = 2
NS = 16
R = 128


def _sc_pooling(Bn, T, D, F):
    n = T // Bn
    half = F // 2
    NB = half // R
    DV = D // L
    nseg_per_core = Bn // NC

    mesh = plsc.VectorSubcoreMesh(
        core_axis_name="c", subcore_axis_name="s", num_cores=NC,
        num_subcores=NS)

    @functools.partial(
        pl.kernel,
        out_type=jax.ShapeDtypeStruct((Bn, D + L), jnp.float32),
        mesh=mesh,
        compiler_params=pltpu.CompilerParams(needs_layout_passes=False),
        scratch_types=[
            pltpu.VMEM((D + 2 * L,), jnp.float32),
            pltpu.VMEM((2, R, D), jnp.float32),
            pltpu.VMEM((R,), jnp.float32),
            pltpu.VMEM((L * L,), jnp.float32),
            pltpu.VMEM((D + L,), jnp.float32),
            pltpu.VMEM((D + L,), jnp.float32),
            pltpu.VMEM((D + L,), jnp.float32),
            pltpu.VMEM_SHARED((NS * 512,), jnp.float32),
            pltpu.SemaphoreType.DMA,
            pltpu.SemaphoreType.DMA,
        ],
    )
    def call(x_hbm, params_hbm, out_hbm, params_v, xbuf, logits_v, pscr,
             acc_ref, pacc_v, outbuf, shex, sem0, sem1):
        c = lax.axis_index("c")
        s = lax.axis_index("s")
        seg = c * nseg_per_core + s // 2
        h = s % 2
        base = seg * n + h * half

        pltpu.sync_copy(params_hbm, params_v)
        lane = lax.iota(jnp.int32, L)
        pcoef_v = params_v[pl.ds(D, L)]
        pcoef = jnp.sum(jnp.where(lane == seg, pcoef_v, 0.0))
        bias = params_v[pl.ds(D + Bn, L)][0]
        lanef = lane.astype(jnp.float32)
        zero16 = jnp.zeros((L,), jnp.float32)
        for j in range(DV):
            acc_ref[pl.ds(L * j, L)] = zero16

        sems = (sem0, sem1)
        pltpu.async_copy(x_hbm.at[pl.ds(base, R), :], xbuf.at[0], sem0)
        pltpu.async_copy(x_hbm.at[pl.ds(base + R, R), :], xbuf.at[1], sem1)

        def process(blk, b, m, s_v):
            xb = xbuf.at[b]
            toff = (h * half + blk * R).astype(jnp.float32)

            def grp_a(g, bm_v):
                rb = g * L
                for q in range(0, L, 8):
                    p = [zero16] * 8
                    for j in range(DV):
                        w = params_v[pl.ds(L * j, L)]
                        for r in range(8):
                            p[r] = p[r] + xb[rb + q + r,
                                             pl.ds(L * j, L)] * w
                    for r in range(8):
                        pscr[pl.ds((q + r) * L, L)] = p[r]
                idxb = lane * L
                dv = plsc.load_gather(pscr, [idxb])
                for cc in range(1, L):
                    dv = dv + plsc.load_gather(pscr, [idxb + cc])
                tloc = lanef + (toff + (g * L).astype(jnp.float32))
                lv = dv + tloc * pcoef + bias
                logits_v[pl.ds(g * L, L)] = lv
                return jnp.maximum(bm_v, lv)

            bmax_v = lax.fori_loop(0, R // L, grp_a,
                                   jnp.full((L,), -1e30, jnp.float32))
            m_new = jnp.maximum(m, jnp.max(bmax_v))
            resc = jnp.exp(jnp.full((L,), m - m_new))
            s_v = s_v * resc
            for j in range(DV):
                acc_ref[pl.ds(L * j, L)] = acc_ref[pl.ds(L * j, L)] * resc

            def grp_b(g, sv):
                rb = g * L
                ev = jnp.exp(logits_v[pl.ds(g * L, L)] - m_new)
                for q in range(0, L, 8):
                    esp = [jnp.full((L,), ev[q + r]) for r in range(8)]
                    for jc in range(0, DV, 8):
                        contrib = []
                        for j in range(jc, jc + 8):
                            pr = [esp[r] * xb[rb + q + r, pl.ds(L * j, L)]
                                  for r in range(8)]
                            s01 = pr[0] + pr[1]
                            s23 = pr[2] + pr[3]
                            s45 = pr[4] + pr[5]
                            s67 = pr[6] + pr[7]
                            contrib.append((s01 + s23) + (s45 + s67))
                        for j in range(jc, jc + 8):
                            plsc.addupdate(acc_ref.at[pl.ds(L * j, L)],
                                           contrib[j - jc])
                return sv + ev

            s_v = lax.fori_loop(0, R // L, grp_b, s_v)
            return m_new, s_v

        def pair(p, carry):
            m, s_v = carry
            for sub in range(2):
                blk = 2 * p + sub
                wait_src = x_hbm.at[pl.ds(0, R), :]
                pltpu.make_async_copy(wait_src, xbuf.at[sub],
                                      sems[sub]).wait()
                m, s_v = process(blk, sub, m, s_v)

                @pl.when(blk + 2 < NB)
                def _():
                    pltpu.async_copy(
                        x_hbm.at[pl.ds(base + (blk + 2) * R, R), :],
                        xbuf.at[sub], sems[sub])
            return m, s_v

        m, s_v = lax.fori_loop(
            0, NB // 2, pair,
            (jnp.float32(-1e30), zero16))

        s_loc = jnp.sum(s_v)
        st = jnp.where(lane == 0, m, jnp.where(lane == 1, s_loc, 0.0))
        acc_ref[pl.ds(D, L)] = st
        pltpu.sync_copy(acc_ref, shex.at[pl.ds(s * 512, D + L)])
        plsc.subcore_barrier()

        @pl.when(h == 0)
        def _():
            pltpu.sync_copy(shex.at[pl.ds((s + 1) * 512, D + L)], pacc_v)
            pst = pacc_v[pl.ds(D, L)]
            m2 = pst[0]
            s2 = pst[1]
            mf = jnp.maximum(m, m2)
            a1 = jnp.exp(jnp.full((L,), m - mf))
            a2 = jnp.exp(jnp.full((L,), m2 - mf))
            for j in range(DV):
                outbuf[pl.ds(L * j, L)] = (
                    acc_ref[pl.ds(L * j, L)] * a1
                    + pacc_v[pl.ds(L * j, L)] * a2)
            sp = a1 * s_loc + a2 * s2
            stv = jnp.where(lane == 0, mf, jnp.where(lane == 1, sp[0], 0.0))
            outbuf[pl.ds(D, L)] = stv
            pltpu.sync_copy(outbuf, out_hbm.at[seg])

    return call


def _tc_tail(Bn, T, D, F):
    n = T // Bn
    RT = n - F

    def body(x_ref, w_ref, p2_ref, stats_ref, acc_ref):
        i = pl.program_id(0)
        xb = x_ref[...]
        wv = w_ref[...]
        pr = p2_ref[pl.ds(i, 1), :]
        pc = pr[0:1, 0:1]
        bb = pr[0:1, 1:2]
        tloc = jax.lax.broadcasted_iota(
            jnp.int32, (RT, 1), 0).astype(jnp.float32) + float(F)
        lg = jnp.dot(xb, wv,
                     preferred_element_type=jnp.float32) + tloc * pc + bb
        m = jnp.max(lg)
        e = jnp.exp(lg - m)
        col = jax.lax.broadcasted_iota(jnp.int32, (1, 128), 1)
        stats_ref[pl.ds(i, 1), :] = jnp.where(
            col == 0, m, jnp.where(col == 1, jnp.sum(e), 0.0))
        acc_ref[pl.ds(i, 1), :] = jnp.sum(xb * e, axis=0, keepdims=True)

    return pl.pallas_call(
        body,
        grid=(Bn,),
        in_specs=[
            pl.BlockSpec((RT, D), lambda i: (i * (n // RT) + F // RT, 0)),
            pl.BlockSpec((D, 1), lambda i: (0, 0)),
            pl.BlockSpec((Bn, 128), lambda i: (0, 0)),
        ],
        out_specs=[
            pl.BlockSpec((Bn, 128), lambda i: (0, 0)),
            pl.BlockSpec((Bn, D), lambda i: (0, 0)),
        ],
        out_shape=[
            jax.ShapeDtypeStruct((Bn, 128), jnp.float32),
            jax.ShapeDtypeStruct((Bn, D), jnp.float32),
        ],
    )


def kernel(x, W, b, slices):
    T, D = x.shape
    Bn = slices.shape[0]
    n = T // Bn
    F = n // 2
    wx = W[0, 1:]
    pcoef = W[0, 0] / slices.astype(jnp.float32)
    pad = jnp.zeros((2 * L - Bn - 1,), jnp.float32)
    params = jnp.concatenate([wx, pcoef, b.astype(jnp.float32), pad])

    params2 = jnp.pad(
        jnp.stack([pcoef, jnp.full((Bn,), b[0], jnp.float32)], axis=1),
        ((0, 0), (0, 126)))
    sc_out = _sc_pooling(Bn, T, D, F)(x, params)
    stats_tc, acc_tc = _tc_tail(Bn, T, D, F)(x, wx[:, None], params2)

    acc_sc = sc_out[:, :D]
    m_sc = sc_out[:, D]
    s_sc = sc_out[:, D + 1]
    m2 = stats_tc[:, 0]
    s2 = stats_tc[:, 1]
    mf = jnp.maximum(m_sc, m2)
    a1 = jnp.exp(m_sc - mf)
    a2 = jnp.exp(m2 - mf)
    denom = s_sc * a1 + s2 * a2
    return (acc_sc * a1[:, None] + acc_tc * a2[:, None]) / denom[:, None]
